# baseline (speedup 1.0000x reference)
import functools

import jax
import jax.numpy as jnp
from jax import lax
from jax.experimental import pallas as pl
from jax.experimental.pallas import tpu as pltpu
from jax.experimental.pallas import tpu_sc as plsc

_N = 10000
_E = 320000
_H = 128
_DE = 9

_NW = 32
_NP = 10240
_RPT = _NP // _NW
_KS = 4000
_NSC = _E // _KS
_KG = 80

_RB = 400
_NB = _N // _RB

_f32 = jnp.float32
_i32 = jnp.int32


def _sc_mesh():
    return plsc.VectorSubcoreMesh(core_axis_name="c", subcore_axis_name="s")


@functools.partial(
    pl.kernel,
    out_type=jax.ShapeDtypeStruct((_NP, 16), _f32),
    mesh=_sc_mesh(),
    compiler_params=pltpu.CompilerParams(needs_layout_passes=False),
    scratch_types=[
        pltpu.VMEM((_N + 16,), _i32),
        pltpu.VMEM((_KS,), _i32),
        pltpu.VMEM((_KS,), _i32),
        pltpu.VMEM((_KS + _KG,), _i32),
        pltpu.VMEM((_KS + _KG,), _i32),
        pltpu.VMEM((_KS + _KG,), _i32),
        pltpu.VMEM((_KS + _KG,), _i32),
        pltpu.VMEM((_KG, _H), _f32),
        pltpu.VMEM((_RPT, 16), _f32),
        pltpu.SemaphoreType.DMA,
    ],
)
def _sc_edge_stats(dst_hbm, src_hbm, epack_hbm, x_hbm, out_hbm,
                   x_v, dst_v, src_v, mdst_v, msrc_v, meid_v, mrow_v, rows_v,
                   acc_v, sem):
    c = lax.axis_index("c")
    s = lax.axis_index("s")
    wid = s * 2 + c
    lo = wid * _RPT
    zeros16 = jnp.zeros((16,), _f32)
    iota = lax.iota(_i32, 16)

    def _zrow(i, carry):
        acc_v[i, :] = zeros16
        return carry

    lax.fori_loop(0, _RPT, _zrow, 0)
    pltpu.sync_copy(x_hbm, x_v.at[pl.ds(0, _N)])

    def _chunk(ci, carry):
        e0 = ci * _KS
        pltpu.sync_copy(dst_hbm.at[pl.ds(e0, _KS)], dst_v)
        pltpu.sync_copy(src_hbm.at[pl.ds(e0, _KS)], src_v)

        def _group(gi, mc):
            d16 = dst_v[pl.ds(gi * 16, 16)]
            s16 = src_v[pl.ds(gi * 16, 16)]
            dl16 = d16 - lo
            mask = (dl16 >= 0) & (dl16 < _RPT)
            plsc.store_compressed(mdst_v.at[pl.ds(mc, 16)], dl16, mask=mask)
            plsc.store_compressed(msrc_v.at[pl.ds(mc, 16)], s16, mask=mask)
            e16 = iota + (e0 + gi * 16)
            plsc.store_compressed(meid_v.at[pl.ds(mc, 16)], e16, mask=mask)
            plsc.store_compressed(mrow_v.at[pl.ds(mc, 16)],
                                  lax.shift_right_logical(e16, 3), mask=mask)
            return mc + jnp.sum(mask.astype(_i32))

        mc = lax.fori_loop(0, _KS // 16, _group, 0)

        zi16 = jnp.zeros((16,), _i32)
        for j in range(_KG // 16):
            mrow_v[pl.ds(mc + j * 16, 16)] = zi16
        ng = (mc + _KG - 1) // _KG
        onehot10 = (iota == 10).astype(_f32)

        def _gblock(gi, carry1):
            pltpu.async_copy(
                epack_hbm.at[mrow_v.at[pl.ds(gi * _KG, _KG)]], rows_v, sem
            ).wait()
            nedge = jnp.minimum(_KG, mc - gi * _KG)

            def _edge(k, carry2):
                kk = gi * _KG + k
                e = meid_v[pl.ds(kk, 16)][0]
                dl = mdst_v[pl.ds(kk, 16)][0]
                sv = msrc_v[pl.ds(kk, 16)][0]
                xv = x_v[pl.ds(sv, 16)][0].astype(_f32)
                off = jnp.bitwise_and(e, 7) * 16
                vals = rows_v[k, pl.ds(off, 16)]
                acc_v[dl, :] = acc_v[dl, :] + vals + xv * onehot10
                return carry2

            lax.fori_loop(0, nedge, _edge, 0)
            return carry1

        lax.fori_loop(0, ng, _gblock, 0)
        return carry

    lax.fori_loop(0, _NSC, _chunk, 0)
    pltpu.sync_copy(acc_v, out_hbm.at[pl.ds(lo, _RPT)])


@functools.partial(
    pl.kernel,
    out_type=jax.ShapeDtypeStruct((_NP, _H), _f32),
    mesh=_sc_mesh(),
    compiler_params=pltpu.CompilerParams(needs_layout_passes=False),
    scratch_types=[
        pltpu.VMEM((_KS,), _i32),
        pltpu.VMEM((_KS,), _i32),
        pltpu.VMEM((_KS + _KG,), _i32),
        pltpu.VMEM((_KS + _KG,), _i32),
        pltpu.VMEM((_KG, _H), _f32),
        pltpu.VMEM((_RPT, _H), _f32),
        pltpu.SemaphoreType.DMA,
    ],
)
def _sc_spmm(dst_hbm, src_hbm, h1_hbm, out_hbm,
             dst_v, src_v, mdst_v, msrc_v, rows_v, acc_v, sem):
    c = lax.axis_index("c")
    s = lax.axis_index("s")
    wid = s * 2 + c
    lo = wid * _RPT
    zeros16 = jnp.zeros((16,), _f32)
    iota = lax.iota(_i32, 16)

    def _zrow(i, carry):
        for j in range(_H // 16):
            acc_v[i, pl.ds(j * 16, 16)] = zeros16
        return carry

    lax.fori_loop(0, _RPT, _zrow, 0)

    def _chunk(ci, carry):
        e0 = ci * _KS
        pltpu.sync_copy(dst_hbm.at[pl.ds(e0, _KS)], dst_v)
        pltpu.sync_copy(src_hbm.at[pl.ds(e0, _KS)], src_v)

        def _group(gi, mc):
            d16 = dst_v[pl.ds(gi * 16, 16)]
            s16 = src_v[pl.ds(gi * 16, 16)]
            dl16 = d16 - lo
            mask = (dl16 >= 0) & (dl16 < _RPT)
            plsc.store_compressed(mdst_v.at[pl.ds(mc, 16)], dl16, mask=mask)
            plsc.store_compressed(msrc_v.at[pl.ds(mc, 16)], s16, mask=mask)
            return mc + jnp.sum(mask.astype(_i32))

        mc = lax.fori_loop(0, _KS // 16, _group, 0)

        zi16 = jnp.zeros((16,), _i32)
        for j in range(_KG // 16):
            msrc_v[pl.ds(mc + j * 16, 16)] = zi16
        ng = (mc + _KG - 1) // _KG

        def _gblock(gi, carry1):
            pltpu.async_copy(
                h1_hbm.at[msrc_v.at[pl.ds(gi * _KG, _KG)]], rows_v, sem
            ).wait()
            nedge = jnp.minimum(_KG, mc - gi * _KG)

            def _edge(k, carry2):
                kk = gi * _KG + k
                dl = mdst_v[pl.ds(kk, 16)][0]
                for j in range(_H // 16):
                    sl = pl.ds(j * 16, 16)
                    acc_v[dl, sl] = acc_v[dl, sl] + rows_v[k, sl]
                return carry2

            lax.fori_loop(0, nedge, _edge, 0)
            return carry1

        lax.fori_loop(0, ng, _gblock, 0)
        return carry

    lax.fori_loop(0, _NSC, _chunk, 0)
    pltpu.sync_copy(acc_v, out_hbm.at[pl.ds(lo, _RPT)])


def _row_spec(cols):
    return pl.BlockSpec((_RB, cols), lambda i: (i, 0))


def _full_spec(shape):
    nd = len(shape)
    return pl.BlockSpec(shape, lambda i, _nd=nd: (0,) * _nd)


def _stats_tail(z, s_ref, q_ref):
    cs = jnp.sum(z, axis=0, keepdims=True)
    cq = jnp.sum(z * z, axis=0, keepdims=True)

    @pl.when(pl.program_id(0) == 0)
    def _():
        s_ref[...] = cs
        q_ref[...] = cq

    @pl.when(pl.program_id(0) != 0)
    def _():
        s_ref[...] = s_ref[...] + cs
        q_ref[...] = q_ref[...] + cq


def _a0_body(acc_ref, xf_ref, emb_ref, sla_ref, We_ref, be_ref,
             W1_ref, b1_ref, z_ref, s_ref, q_ref):
    acc = acc_ref[...]
    degt = acc[:, 9:10] + 1.0
    St = acc[:, 10:11] + xf_ref[...]
    e0 = emb_ref[0:1, :]
    d10 = emb_ref[1:2, :] - e0
    node = degt * e0 + St * d10
    at = acc + sla_ref[...]
    edgep = jnp.dot(at, We_ref[...], preferred_element_type=_f32)
    edgep = edgep + degt * be_ref[...]
    agg = jnp.concatenate([node, edgep], axis=1)
    z = jnp.dot(agg, W1_ref[...], preferred_element_type=_f32) + b1_ref[...]
    z_ref[...] = z
    _stats_tail(z, s_ref, q_ref)


def _a1_body(p_ref, h1_ref, acc_ref, sla_ref, We_ref,
             be_ref, W1_ref, b1_ref, z_ref, s_ref, q_ref):
    acc = acc_ref[...]
    degt = acc[:, 9:10] + 1.0
    node = p_ref[...] + h1_ref[...]
    at = acc + sla_ref[...]
    edgep = jnp.dot(at, We_ref[...], preferred_element_type=_f32)
    edgep = edgep + degt * be_ref[...]
    agg = jnp.concatenate([node, edgep], axis=1)
    z = jnp.dot(agg, W1_ref[...], preferred_element_type=_f32) + b1_ref[...]
    z_ref[...] = z
    _stats_tail(z, s_ref, q_ref)


def _b_body(final_relu, z_ref, s_ref, q_ref, g_ref, bt_ref, W2_ref, b2_ref,
            o_ref):
    inv_n = _f32(1.0 / _N)
    m = s_ref[...] * inv_n
    var = q_ref[...] * inv_n - m * m
    rstd = lax.rsqrt(var + 1e-5)
    zn = (z_ref[...] - m) * (rstd * g_ref[...]) + bt_ref[...]
    r = jnp.maximum(zn, 0.0)
    o = jnp.dot(r, W2_ref[...], preferred_element_type=_f32) + b2_ref[...]
    if final_relu:
        o = jnp.maximum(o, 0.0)
    o_ref[...] = o


def _tc_a0(acc, xf, emb, sla16, We16, be, W1, b1):
    return pl.pallas_call(
        _a0_body,
        grid=(_NB,),
        in_specs=[
            _row_spec(16), _row_spec(1),
            _full_spec((2, _H)), _full_spec((1, 16)), _full_spec((16, _H)),
            _full_spec((1, _H)), _full_spec((2 * _H, 2 * _H)),
            _full_spec((1, 2 * _H)),
        ],
        out_specs=[
            _row_spec(2 * _H),
            _full_spec((1, 2 * _H)), _full_spec((1, 2 * _H)),
        ],
        out_shape=[
            jax.ShapeDtypeStruct((_N, 2 * _H), _f32),
            jax.ShapeDtypeStruct((1, 2 * _H), _f32),
            jax.ShapeDtypeStruct((1, 2 * _H), _f32),
        ],
    )(acc, xf, emb, sla16, We16, be, W1, b1)


def _tc_a1(p, h1, acc, sla16, We16, be, W1, b1):
    return pl.pallas_call(
        _a1_body,
        grid=(_NB,),
        in_specs=[
            _row_spec(_H), _row_spec(_H), _row_spec(16),
            _full_spec((1, 16)), _full_spec((16, _H)), _full_spec((1, _H)),
            _full_spec((2 * _H, 2 * _H)), _full_spec((1, 2 * _H)),
        ],
        out_specs=[
            _row_spec(2 * _H),
            _full_spec((1, 2 * _H)), _full_spec((1, 2 * _H)),
        ],
        out_shape=[
            jax.ShapeDtypeStruct((_N, 2 * _H), _f32),
            jax.ShapeDtypeStruct((1, 2 * _H), _f32),
            jax.ShapeDtypeStruct((1, 2 * _H), _f32),
        ],
    )(p, h1, acc, sla16, We16, be, W1, b1)


def _tc_b(z, ssum, sq, g, bt, W2, b2, final_relu):
    return pl.pallas_call(
        functools.partial(_b_body, final_relu),
        grid=(_NB,),
        in_specs=[
            _row_spec(2 * _H),
            _full_spec((1, 2 * _H)), _full_spec((1, 2 * _H)),
            _full_spec((1, 2 * _H)), _full_spec((1, 2 * _H)),
            _full_spec((2 * _H, _H)), _full_spec((1, _H)),
        ],
        out_specs=_row_spec(_H),
        out_shape=jax.ShapeDtypeStruct((_N, _H), _f32),
    )(z, ssum, sq, g, bt, W2, b2)


def kernel(x, edge_index, edge_attr, self_loop_index, self_loop_type, embed,
           We0, be0, W10, b10, g0, bt0, W20, b20,
           We1, be1, W11, b11, g1, bt1, W21, b21):
    xi = x.astype(_i32)
    src = edge_index[1].astype(_i32)
    dst = edge_index[0].astype(_i32)
    eap = jnp.concatenate(
        [edge_attr.astype(_f32),
         jnp.ones((_E, 1), _f32),
         jnp.zeros((_E, 6), _f32)], axis=1)

    epack = eap.reshape(_E // 8, 8 * 16)
    acc16 = _sc_edge_stats(dst, src, epack, xi)
    acc = acc16[:_N]

    xf = xi.astype(_f32).reshape(_N, 1)
    sltf = jnp.asarray(self_loop_type, _f32)
    sla16 = jnp.zeros((1, 16), _f32).at[0, self_loop_index].set(sltf)
    zpad = jnp.zeros((16 - _DE, _H), _f32)
    We16_0 = jnp.concatenate([We0.astype(_f32), zpad], axis=0)
    We16_1 = jnp.concatenate([We1.astype(_f32), zpad], axis=0)

    z0, s0, q0 = _tc_a0(acc, xf, embed, sla16, We16_0,
                        be0.reshape(1, _H), W10, b10.reshape(1, 2 * _H))
    h1 = _tc_b(z0, s0, q0, g0.reshape(1, -1), bt0.reshape(1, -1),
               W20, b20.reshape(1, -1), final_relu=True)

    p = _sc_spmm(dst, src, h1)

    z1, s1, q1 = _tc_a1(p[:_N], h1, acc, sla16, We16_1,
                        be1.reshape(1, _H), W11, b11.reshape(1, 2 * _H))
    out = _tc_b(z1, s1, q1, g1.reshape(1, -1), bt1.reshape(1, -1),
                W21, b21.reshape(1, -1), final_relu=False)
    return out


# speedup vs baseline: 1.3149x; 1.3149x over previous
import functools

import jax
import jax.numpy as jnp
from jax import lax
from jax.experimental import pallas as pl
from jax.experimental.pallas import tpu as pltpu
from jax.experimental.pallas import tpu_sc as plsc

_N = 10000
_E = 320000
_H = 128
_DE = 9

_NW = 32
_NP = 10240
_RPT = _NP // _NW
_KS = 8000
_NSC = _E // _KS
_KG = 80

_RB = 400
_NB = _N // _RB

_f32 = jnp.float32
_i32 = jnp.int32


def _sc_mesh():
    return plsc.VectorSubcoreMesh(core_axis_name="c", subcore_axis_name="s")


@functools.partial(
    pl.kernel,
    out_type=jax.ShapeDtypeStruct((_NP, 16), _f32),
    mesh=_sc_mesh(),
    compiler_params=pltpu.CompilerParams(needs_layout_passes=False),
    scratch_types=[
        pltpu.VMEM((_N + 16,), _i32),
        pltpu.VMEM((_KS,), _i32),
        pltpu.VMEM((_KS,), _i32),
        pltpu.VMEM((_KS + _KG,), _i32),
        pltpu.VMEM((_KS + _KG,), _i32),
        pltpu.VMEM((_KS + _KG,), _i32),
        pltpu.VMEM((_KS + _KG,), _i32),
        pltpu.VMEM((_KG, _H), _f32),
        pltpu.VMEM((_RPT, 16), _f32),
        pltpu.SemaphoreType.DMA,
    ],
)
def _sc_edge_stats(dst_hbm, src_hbm, epack_hbm, x_hbm, out_hbm,
                   x_v, dst_v, src_v, mdst_v, msrc_v, meid_v, mrow_v, rows_v,
                   acc_v, sem):
    c = lax.axis_index("c")
    s = lax.axis_index("s")
    wid = s * 2 + c
    lo = wid * _RPT
    zeros16 = jnp.zeros((16,), _f32)
    iota = lax.iota(_i32, 16)

    def _zrow(i, carry):
        acc_v[i, :] = zeros16
        return carry

    lax.fori_loop(0, _RPT, _zrow, 0)
    pltpu.sync_copy(x_hbm, x_v.at[pl.ds(0, _N)])

    def _chunk(ci, carry):
        e0 = ci * _KS
        pltpu.sync_copy(dst_hbm.at[pl.ds(e0, _KS)], dst_v)
        pltpu.sync_copy(src_hbm.at[pl.ds(e0, _KS)], src_v)

        def _group(gi, mc):
            d16 = dst_v[pl.ds(gi * 16, 16)]
            s16 = src_v[pl.ds(gi * 16, 16)]
            dl16 = d16 - lo
            mask = (dl16 >= 0) & (dl16 < _RPT)
            plsc.store_compressed(mdst_v.at[pl.ds(mc, 16)], dl16, mask=mask)
            plsc.store_compressed(msrc_v.at[pl.ds(mc, 16)], s16, mask=mask)
            e16 = iota + (e0 + gi * 16)
            plsc.store_compressed(meid_v.at[pl.ds(mc, 16)], e16, mask=mask)
            plsc.store_compressed(mrow_v.at[pl.ds(mc, 16)],
                                  lax.shift_right_logical(e16, 3), mask=mask)
            return mc + plsc.all_reduce_population_count(mask)[0]

        mc = lax.fori_loop(0, _KS // 16, _group, 0)

        zi16 = jnp.zeros((16,), _i32)
        for j in range(_KG // 16):
            mrow_v[pl.ds(mc + j * 16, 16)] = zi16
        ng = (mc + _KG - 1) // _KG
        onehot10 = (iota == 10).astype(_f32)

        def _gblock(gi, carry1):
            pltpu.async_copy(
                epack_hbm.at[mrow_v.at[pl.ds(gi * _KG, _KG)]], rows_v, sem
            ).wait()
            nedge = jnp.minimum(_KG, mc - gi * _KG)

            def _edge(k, carry2):
                kk = gi * _KG + k
                e = meid_v[pl.ds(kk, 16)][0]
                dl = mdst_v[pl.ds(kk, 16)][0]
                sv = msrc_v[pl.ds(kk, 16)][0]
                xv = x_v[pl.ds(sv, 16)][0].astype(_f32)
                off = jnp.bitwise_and(e, 7) * 16
                vals = rows_v[k, pl.ds(off, 16)]
                acc_v[dl, :] = acc_v[dl, :] + vals + xv * onehot10
                return carry2

            lax.fori_loop(0, nedge, _edge, 0)
            return carry1

        lax.fori_loop(0, ng, _gblock, 0)
        return carry

    lax.fori_loop(0, _NSC, _chunk, 0)
    pltpu.sync_copy(acc_v, out_hbm.at[pl.ds(lo, _RPT)])


@functools.partial(
    pl.kernel,
    out_type=jax.ShapeDtypeStruct((_NP, _H), _f32),
    mesh=_sc_mesh(),
    compiler_params=pltpu.CompilerParams(needs_layout_passes=False),
    scratch_types=[
        pltpu.VMEM((_KS,), _i32),
        pltpu.VMEM((_KS,), _i32),
        pltpu.VMEM((_KS + _KG,), _i32),
        pltpu.VMEM((_KS + _KG,), _i32),
        pltpu.VMEM((_KG, _H), _f32),
        pltpu.VMEM((_RPT, _H), _f32),
        pltpu.SemaphoreType.DMA,
    ],
)
def _sc_spmm(dst_hbm, src_hbm, h1_hbm, out_hbm,
             dst_v, src_v, mdst_v, msrc_v, rows_v, acc_v, sem):
    c = lax.axis_index("c")
    s = lax.axis_index("s")
    wid = s * 2 + c
    lo = wid * _RPT
    zeros16 = jnp.zeros((16,), _f32)
    iota = lax.iota(_i32, 16)

    def _zrow(i, carry):
        for j in range(_H // 16):
            acc_v[i, pl.ds(j * 16, 16)] = zeros16
        return carry

    lax.fori_loop(0, _RPT, _zrow, 0)

    def _chunk(ci, carry):
        e0 = ci * _KS
        pltpu.sync_copy(dst_hbm.at[pl.ds(e0, _KS)], dst_v)
        pltpu.sync_copy(src_hbm.at[pl.ds(e0, _KS)], src_v)

        def _group(gi, mc):
            d16 = dst_v[pl.ds(gi * 16, 16)]
            s16 = src_v[pl.ds(gi * 16, 16)]
            dl16 = d16 - lo
            mask = (dl16 >= 0) & (dl16 < _RPT)
            plsc.store_compressed(mdst_v.at[pl.ds(mc, 16)], dl16, mask=mask)
            plsc.store_compressed(msrc_v.at[pl.ds(mc, 16)], s16, mask=mask)
            return mc + plsc.all_reduce_population_count(mask)[0]

        mc = lax.fori_loop(0, _KS // 16, _group, 0)

        zi16 = jnp.zeros((16,), _i32)
        for j in range(_KG // 16):
            msrc_v[pl.ds(mc + j * 16, 16)] = zi16
        ng = (mc + _KG - 1) // _KG

        def _gblock(gi, carry1):
            pltpu.async_copy(
                h1_hbm.at[msrc_v.at[pl.ds(gi * _KG, _KG)]], rows_v, sem
            ).wait()
            nedge = jnp.minimum(_KG, mc - gi * _KG)

            def _edge(k, carry2):
                kk = gi * _KG + k
                dl = mdst_v[pl.ds(kk, 16)][0]
                vals = [rows_v[k, pl.ds(j * 16, 16)]
                        for j in range(_H // 16)]
                avs = [acc_v[dl, pl.ds(j * 16, 16)]
                       for j in range(_H // 16)]
                for j in range(_H // 16):
                    acc_v[dl, pl.ds(j * 16, 16)] = avs[j] + vals[j]
                return carry2

            lax.fori_loop(0, nedge, _edge, 0)
            return carry1

        lax.fori_loop(0, ng, _gblock, 0)
        return carry

    lax.fori_loop(0, _NSC, _chunk, 0)
    pltpu.sync_copy(acc_v, out_hbm.at[pl.ds(lo, _RPT)])


def _row_spec(cols):
    return pl.BlockSpec((_RB, cols), lambda i: (i, 0))


def _full_spec(shape):
    nd = len(shape)
    return pl.BlockSpec(shape, lambda i, _nd=nd: (0,) * _nd)


def _stats_tail(z, s_ref, q_ref):
    cs = jnp.sum(z, axis=0, keepdims=True)
    cq = jnp.sum(z * z, axis=0, keepdims=True)

    @pl.when(pl.program_id(0) == 0)
    def _():
        s_ref[...] = cs
        q_ref[...] = cq

    @pl.when(pl.program_id(0) != 0)
    def _():
        s_ref[...] = s_ref[...] + cs
        q_ref[...] = q_ref[...] + cq


def _a0_body(acc_ref, xf_ref, emb_ref, sla_ref, We_ref, be_ref,
             W1_ref, b1_ref, z_ref, s_ref, q_ref):
    acc = acc_ref[...]
    degt = acc[:, 9:10] + 1.0
    St = acc[:, 10:11] + xf_ref[...]
    e0 = emb_ref[0:1, :]
    d10 = emb_ref[1:2, :] - e0
    node = degt * e0 + St * d10
    at = acc + sla_ref[...]
    edgep = jnp.dot(at, We_ref[...], preferred_element_type=_f32)
    edgep = edgep + degt * be_ref[...]
    agg = jnp.concatenate([node, edgep], axis=1)
    z = jnp.dot(agg, W1_ref[...], preferred_element_type=_f32) + b1_ref[...]
    z_ref[...] = z
    _stats_tail(z, s_ref, q_ref)


def _a1_body(p_ref, h1_ref, acc_ref, sla_ref, We_ref,
             be_ref, W1_ref, b1_ref, z_ref, s_ref, q_ref):
    acc = acc_ref[...]
    degt = acc[:, 9:10] + 1.0
    node = p_ref[...] + h1_ref[...]
    at = acc + sla_ref[...]
    edgep = jnp.dot(at, We_ref[...], preferred_element_type=_f32)
    edgep = edgep + degt * be_ref[...]
    agg = jnp.concatenate([node, edgep], axis=1)
    z = jnp.dot(agg, W1_ref[...], preferred_element_type=_f32) + b1_ref[...]
    z_ref[...] = z
    _stats_tail(z, s_ref, q_ref)


def _b_body(final_relu, z_ref, s_ref, q_ref, g_ref, bt_ref, W2_ref, b2_ref,
            o_ref):
    inv_n = _f32(1.0 / _N)
    m = s_ref[...] * inv_n
    var = q_ref[...] * inv_n - m * m
    rstd = lax.rsqrt(var + 1e-5)
    zn = (z_ref[...] - m) * (rstd * g_ref[...]) + bt_ref[...]
    r = jnp.maximum(zn, 0.0)
    o = jnp.dot(r, W2_ref[...], preferred_element_type=_f32) + b2_ref[...]
    if final_relu:
        o = jnp.maximum(o, 0.0)
    o_ref[...] = o


def _tc_a0(acc, xf, emb, sla16, We16, be, W1, b1):
    return pl.pallas_call(
        _a0_body,
        grid=(_NB,),
        in_specs=[
            _row_spec(16), _row_spec(1),
            _full_spec((2, _H)), _full_spec((1, 16)), _full_spec((16, _H)),
            _full_spec((1, _H)), _full_spec((2 * _H, 2 * _H)),
            _full_spec((1, 2 * _H)),
        ],
        out_specs=[
            _row_spec(2 * _H),
            _full_spec((1, 2 * _H)), _full_spec((1, 2 * _H)),
        ],
        out_shape=[
            jax.ShapeDtypeStruct((_N, 2 * _H), _f32),
            jax.ShapeDtypeStruct((1, 2 * _H), _f32),
            jax.ShapeDtypeStruct((1, 2 * _H), _f32),
        ],
    )(acc, xf, emb, sla16, We16, be, W1, b1)


def _tc_a1(p, h1, acc, sla16, We16, be, W1, b1):
    return pl.pallas_call(
        _a1_body,
        grid=(_NB,),
        in_specs=[
            _row_spec(_H), _row_spec(_H), _row_spec(16),
            _full_spec((1, 16)), _full_spec((16, _H)), _full_spec((1, _H)),
            _full_spec((2 * _H, 2 * _H)), _full_spec((1, 2 * _H)),
        ],
        out_specs=[
            _row_spec(2 * _H),
            _full_spec((1, 2 * _H)), _full_spec((1, 2 * _H)),
        ],
        out_shape=[
            jax.ShapeDtypeStruct((_N, 2 * _H), _f32),
            jax.ShapeDtypeStruct((1, 2 * _H), _f32),
            jax.ShapeDtypeStruct((1, 2 * _H), _f32),
        ],
    )(p, h1, acc, sla16, We16, be, W1, b1)


def _tc_b(z, ssum, sq, g, bt, W2, b2, final_relu):
    return pl.pallas_call(
        functools.partial(_b_body, final_relu),
        grid=(_NB,),
        in_specs=[
            _row_spec(2 * _H),
            _full_spec((1, 2 * _H)), _full_spec((1, 2 * _H)),
            _full_spec((1, 2 * _H)), _full_spec((1, 2 * _H)),
            _full_spec((2 * _H, _H)), _full_spec((1, _H)),
        ],
        out_specs=_row_spec(_H),
        out_shape=jax.ShapeDtypeStruct((_N, _H), _f32),
    )(z, ssum, sq, g, bt, W2, b2)


def kernel(x, edge_index, edge_attr, self_loop_index, self_loop_type, embed,
           We0, be0, W10, b10, g0, bt0, W20, b20,
           We1, be1, W11, b11, g1, bt1, W21, b21):
    xi = x.astype(_i32)
    src = edge_index[1].astype(_i32)
    dst = edge_index[0].astype(_i32)
    eap = jnp.concatenate(
        [edge_attr.astype(_f32),
         jnp.ones((_E, 1), _f32),
         jnp.zeros((_E, 6), _f32)], axis=1)

    epack = eap.reshape(_E // 8, 8 * 16)
    acc16 = _sc_edge_stats(dst, src, epack, xi)
    acc = acc16[:_N]

    xf = xi.astype(_f32).reshape(_N, 1)
    sltf = jnp.asarray(self_loop_type, _f32)
    sla16 = jnp.zeros((1, 16), _f32).at[0, self_loop_index].set(sltf)
    zpad = jnp.zeros((16 - _DE, _H), _f32)
    We16_0 = jnp.concatenate([We0.astype(_f32), zpad], axis=0)
    We16_1 = jnp.concatenate([We1.astype(_f32), zpad], axis=0)

    z0, s0, q0 = _tc_a0(acc, xf, embed, sla16, We16_0,
                        be0.reshape(1, _H), W10, b10.reshape(1, 2 * _H))
    h1 = _tc_b(z0, s0, q0, g0.reshape(1, -1), bt0.reshape(1, -1),
               W20, b20.reshape(1, -1), final_relu=True)

    p = _sc_spmm(dst, src, h1)

    z1, s1, q1 = _tc_a1(p[:_N], h1, acc, sla16, We16_1,
                        be1.reshape(1, _H), W11, b11.reshape(1, 2 * _H))
    out = _tc_b(z1, s1, q1, g1.reshape(1, -1), bt1.reshape(1, -1),
                W21, b21.reshape(1, -1), final_relu=False)
    return out


# speedup vs baseline: 1.3202x; 1.0041x over previous
import functools

import jax
import jax.numpy as jnp
from jax import lax
from jax.experimental import pallas as pl
from jax.experimental.pallas import tpu as pltpu
from jax.experimental.pallas import tpu_sc as plsc

_N = 10000
_E = 320000
_H = 128
_DE = 9

_NW = 32
_NP = 10240
_RPT = _NP // _NW
_KS = 8000
_NSC = _E // _KS
_KG = 80

_RB = 400
_NB = _N // _RB

_f32 = jnp.float32
_i32 = jnp.int32


def _sc_mesh():
    return plsc.VectorSubcoreMesh(core_axis_name="c", subcore_axis_name="s")


@functools.partial(
    pl.kernel,
    out_type=jax.ShapeDtypeStruct((_NP, 16), _f32),
    mesh=_sc_mesh(),
    compiler_params=pltpu.CompilerParams(needs_layout_passes=False),
    scratch_types=[
        pltpu.VMEM((_N + 16,), _i32),
        pltpu.VMEM((_KS,), _i32),
        pltpu.VMEM((_KS,), _i32),
        pltpu.VMEM((_KS + _KG,), _i32),
        pltpu.VMEM((_KS + _KG,), _i32),
        pltpu.VMEM((_KS + _KG,), _i32),
        pltpu.VMEM((_KS + _KG,), _i32),
        pltpu.VMEM((2, _KG, _H), _f32),
        pltpu.VMEM((_RPT, 16), _f32),
        pltpu.SemaphoreType.DMA((2,)),
    ],
)
def _sc_edge_stats(dst_hbm, src_hbm, epack_hbm, x_hbm, out_hbm,
                   x_v, dst_v, src_v, mdst_v, msrc_v, meid_v, mrow_v, rows_v,
                   acc_v, sem):
    c = lax.axis_index("c")
    s = lax.axis_index("s")
    wid = s * 2 + c
    lo = wid * _RPT
    zeros16 = jnp.zeros((16,), _f32)
    iota = lax.iota(_i32, 16)

    def _zrow(i, carry):
        acc_v[i, :] = zeros16
        return carry

    lax.fori_loop(0, _RPT, _zrow, 0)
    pltpu.sync_copy(x_hbm, x_v.at[pl.ds(0, _N)])

    def _chunk(ci, carry):
        e0 = ci * _KS
        pltpu.sync_copy(dst_hbm.at[pl.ds(e0, _KS)], dst_v)
        pltpu.sync_copy(src_hbm.at[pl.ds(e0, _KS)], src_v)

        def _group(gi, mc):
            d16 = dst_v[pl.ds(gi * 16, 16)]
            s16 = src_v[pl.ds(gi * 16, 16)]
            dl16 = d16 - lo
            mask = (dl16 >= 0) & (dl16 < _RPT)
            plsc.store_compressed(mdst_v.at[pl.ds(mc, 16)], dl16, mask=mask)
            plsc.store_compressed(msrc_v.at[pl.ds(mc, 16)], s16, mask=mask)
            e16 = iota + (e0 + gi * 16)
            plsc.store_compressed(meid_v.at[pl.ds(mc, 16)], e16, mask=mask)
            plsc.store_compressed(mrow_v.at[pl.ds(mc, 16)],
                                  lax.shift_right_logical(e16, 3), mask=mask)
            return mc + plsc.all_reduce_population_count(mask)[0]

        mc = lax.fori_loop(0, _KS // 16, _group, 0)

        zi16 = jnp.zeros((16,), _i32)
        for j in range(_KG // 16):
            mrow_v[pl.ds(mc + j * 16, 16)] = zi16
        ng = (mc + _KG - 1) // _KG
        onehot10 = (iota == 10).astype(_f32)

        @pl.when(ng > 0)
        def _():
            pltpu.async_copy(
                epack_hbm.at[mrow_v.at[pl.ds(0, _KG)]], rows_v.at[0],
                sem.at[0])

        def _gblock(gi, carry1):
            p = jnp.bitwise_and(gi, 1)
            pltpu.make_async_copy(
                epack_hbm.at[mrow_v.at[pl.ds(gi * _KG, _KG)]], rows_v.at[p],
                sem.at[p]).wait()

            @pl.when(gi + 1 < ng)
            def _():
                pn = 1 - p
                pltpu.async_copy(
                    epack_hbm.at[mrow_v.at[pl.ds((gi + 1) * _KG, _KG)]],
                    rows_v.at[pn], sem.at[pn])

            nedge = jnp.minimum(_KG, mc - gi * _KG)

            def _edge(k, carry2):
                kk = gi * _KG + k
                e = meid_v[pl.ds(kk, 16)][0]
                dl = mdst_v[pl.ds(kk, 16)][0]
                sv = msrc_v[pl.ds(kk, 16)][0]
                xv = x_v[pl.ds(sv, 16)][0].astype(_f32)
                off = jnp.bitwise_and(e, 7) * 16
                vals = rows_v[p, k, pl.ds(off, 16)]
                acc_v[dl, :] = acc_v[dl, :] + vals + xv * onehot10
                return carry2

            lax.fori_loop(0, nedge, _edge, 0)
            return carry1

        lax.fori_loop(0, ng, _gblock, 0)
        return carry

    lax.fori_loop(0, _NSC, _chunk, 0)
    pltpu.sync_copy(acc_v, out_hbm.at[pl.ds(lo, _RPT)])


@functools.partial(
    pl.kernel,
    out_type=jax.ShapeDtypeStruct((_NP, _H), _f32),
    mesh=_sc_mesh(),
    compiler_params=pltpu.CompilerParams(needs_layout_passes=False),
    scratch_types=[
        pltpu.VMEM((_KS,), _i32),
        pltpu.VMEM((_KS,), _i32),
        pltpu.VMEM((_KS + _KG,), _i32),
        pltpu.VMEM((_KS + _KG,), _i32),
        pltpu.VMEM((2, _KG, _H), _f32),
        pltpu.VMEM((_RPT, _H), _f32),
        pltpu.SemaphoreType.DMA((2,)),
    ],
)
def _sc_spmm(dst_hbm, src_hbm, h1_hbm, out_hbm,
             dst_v, src_v, mdst_v, msrc_v, rows_v, acc_v, sem):
    c = lax.axis_index("c")
    s = lax.axis_index("s")
    wid = s * 2 + c
    lo = wid * _RPT
    zeros16 = jnp.zeros((16,), _f32)
    iota = lax.iota(_i32, 16)

    def _zrow(i, carry):
        for j in range(_H // 16):
            acc_v[i, pl.ds(j * 16, 16)] = zeros16
        return carry

    lax.fori_loop(0, _RPT, _zrow, 0)

    def _chunk(ci, carry):
        e0 = ci * _KS
        pltpu.sync_copy(dst_hbm.at[pl.ds(e0, _KS)], dst_v)
        pltpu.sync_copy(src_hbm.at[pl.ds(e0, _KS)], src_v)

        def _group(gi, mc):
            d16 = dst_v[pl.ds(gi * 16, 16)]
            s16 = src_v[pl.ds(gi * 16, 16)]
            dl16 = d16 - lo
            mask = (dl16 >= 0) & (dl16 < _RPT)
            plsc.store_compressed(mdst_v.at[pl.ds(mc, 16)], dl16, mask=mask)
            plsc.store_compressed(msrc_v.at[pl.ds(mc, 16)], s16, mask=mask)
            return mc + plsc.all_reduce_population_count(mask)[0]

        mc = lax.fori_loop(0, _KS // 16, _group, 0)

        zi16 = jnp.zeros((16,), _i32)
        for j in range(_KG // 16):
            msrc_v[pl.ds(mc + j * 16, 16)] = zi16
        ng = (mc + _KG - 1) // _KG

        @pl.when(ng > 0)
        def _():
            pltpu.async_copy(
                h1_hbm.at[msrc_v.at[pl.ds(0, _KG)]], rows_v.at[0], sem.at[0])

        def _gblock(gi, carry1):
            p = jnp.bitwise_and(gi, 1)
            pltpu.make_async_copy(
                h1_hbm.at[msrc_v.at[pl.ds(gi * _KG, _KG)]], rows_v.at[p],
                sem.at[p]).wait()

            @pl.when(gi + 1 < ng)
            def _():
                pn = 1 - p
                pltpu.async_copy(
                    h1_hbm.at[msrc_v.at[pl.ds((gi + 1) * _KG, _KG)]],
                    rows_v.at[pn], sem.at[pn])

            nedge = jnp.minimum(_KG, mc - gi * _KG)

            def _edge(k, carry2):
                kk = gi * _KG + k
                dl = mdst_v[pl.ds(kk, 16)][0]
                vals = [rows_v[p, k, pl.ds(j * 16, 16)]
                        for j in range(_H // 16)]
                avs = [acc_v[dl, pl.ds(j * 16, 16)]
                       for j in range(_H // 16)]
                for j in range(_H // 16):
                    acc_v[dl, pl.ds(j * 16, 16)] = avs[j] + vals[j]
                return carry2

            lax.fori_loop(0, nedge, _edge, 0)
            return carry1

        lax.fori_loop(0, ng, _gblock, 0)
        return carry

    lax.fori_loop(0, _NSC, _chunk, 0)
    pltpu.sync_copy(acc_v, out_hbm.at[pl.ds(lo, _RPT)])


def _row_spec(cols):
    return pl.BlockSpec((_RB, cols), lambda i: (i, 0))


def _full_spec(shape):
    nd = len(shape)
    return pl.BlockSpec(shape, lambda i, _nd=nd: (0,) * _nd)


def _stats_tail(z, s_ref, q_ref):
    cs = jnp.sum(z, axis=0, keepdims=True)
    cq = jnp.sum(z * z, axis=0, keepdims=True)

    @pl.when(pl.program_id(0) == 0)
    def _():
        s_ref[...] = cs
        q_ref[...] = cq

    @pl.when(pl.program_id(0) != 0)
    def _():
        s_ref[...] = s_ref[...] + cs
        q_ref[...] = q_ref[...] + cq


def _a0_body(acc_ref, xf_ref, emb_ref, sla_ref, We_ref, be_ref,
             W1_ref, b1_ref, z_ref, s_ref, q_ref):
    acc = acc_ref[...]
    degt = acc[:, 9:10] + 1.0
    St = acc[:, 10:11] + xf_ref[...]
    e0 = emb_ref[0:1, :]
    d10 = emb_ref[1:2, :] - e0
    node = degt * e0 + St * d10
    at = acc + sla_ref[...]
    edgep = jnp.dot(at, We_ref[...], preferred_element_type=_f32)
    edgep = edgep + degt * be_ref[...]
    agg = jnp.concatenate([node, edgep], axis=1)
    z = jnp.dot(agg, W1_ref[...], preferred_element_type=_f32) + b1_ref[...]
    z_ref[...] = z
    _stats_tail(z, s_ref, q_ref)


def _a1_body(p_ref, h1_ref, acc_ref, sla_ref, We_ref,
             be_ref, W1_ref, b1_ref, z_ref, s_ref, q_ref):
    acc = acc_ref[...]
    degt = acc[:, 9:10] + 1.0
    node = p_ref[...] + h1_ref[...]
    at = acc + sla_ref[...]
    edgep = jnp.dot(at, We_ref[...], preferred_element_type=_f32)
    edgep = edgep + degt * be_ref[...]
    agg = jnp.concatenate([node, edgep], axis=1)
    z = jnp.dot(agg, W1_ref[...], preferred_element_type=_f32) + b1_ref[...]
    z_ref[...] = z
    _stats_tail(z, s_ref, q_ref)


def _b_body(final_relu, z_ref, s_ref, q_ref, g_ref, bt_ref, W2_ref, b2_ref,
            o_ref):
    inv_n = _f32(1.0 / _N)
    m = s_ref[...] * inv_n
    var = q_ref[...] * inv_n - m * m
    rstd = lax.rsqrt(var + 1e-5)
    zn = (z_ref[...] - m) * (rstd * g_ref[...]) + bt_ref[...]
    r = jnp.maximum(zn, 0.0)
    o = jnp.dot(r, W2_ref[...], preferred_element_type=_f32) + b2_ref[...]
    if final_relu:
        o = jnp.maximum(o, 0.0)
    o_ref[...] = o


def _tc_a0(acc, xf, emb, sla16, We16, be, W1, b1):
    return pl.pallas_call(
        _a0_body,
        grid=(_NB,),
        in_specs=[
            _row_spec(16), _row_spec(1),
            _full_spec((2, _H)), _full_spec((1, 16)), _full_spec((16, _H)),
            _full_spec((1, _H)), _full_spec((2 * _H, 2 * _H)),
            _full_spec((1, 2 * _H)),
        ],
        out_specs=[
            _row_spec(2 * _H),
            _full_spec((1, 2 * _H)), _full_spec((1, 2 * _H)),
        ],
        out_shape=[
            jax.ShapeDtypeStruct((_N, 2 * _H), _f32),
            jax.ShapeDtypeStruct((1, 2 * _H), _f32),
            jax.ShapeDtypeStruct((1, 2 * _H), _f32),
        ],
    )(acc, xf, emb, sla16, We16, be, W1, b1)


def _tc_a1(p, h1, acc, sla16, We16, be, W1, b1):
    return pl.pallas_call(
        _a1_body,
        grid=(_NB,),
        in_specs=[
            _row_spec(_H), _row_spec(_H), _row_spec(16),
            _full_spec((1, 16)), _full_spec((16, _H)), _full_spec((1, _H)),
            _full_spec((2 * _H, 2 * _H)), _full_spec((1, 2 * _H)),
        ],
        out_specs=[
            _row_spec(2 * _H),
            _full_spec((1, 2 * _H)), _full_spec((1, 2 * _H)),
        ],
        out_shape=[
            jax.ShapeDtypeStruct((_N, 2 * _H), _f32),
            jax.ShapeDtypeStruct((1, 2 * _H), _f32),
            jax.ShapeDtypeStruct((1, 2 * _H), _f32),
        ],
    )(p, h1, acc, sla16, We16, be, W1, b1)


def _tc_b(z, ssum, sq, g, bt, W2, b2, final_relu):
    return pl.pallas_call(
        functools.partial(_b_body, final_relu),
        grid=(_NB,),
        in_specs=[
            _row_spec(2 * _H),
            _full_spec((1, 2 * _H)), _full_spec((1, 2 * _H)),
            _full_spec((1, 2 * _H)), _full_spec((1, 2 * _H)),
            _full_spec((2 * _H, _H)), _full_spec((1, _H)),
        ],
        out_specs=_row_spec(_H),
        out_shape=jax.ShapeDtypeStruct((_N, _H), _f32),
    )(z, ssum, sq, g, bt, W2, b2)


def kernel(x, edge_index, edge_attr, self_loop_index, self_loop_type, embed,
           We0, be0, W10, b10, g0, bt0, W20, b20,
           We1, be1, W11, b11, g1, bt1, W21, b21):
    xi = x.astype(_i32)
    src = edge_index[1].astype(_i32)
    dst = edge_index[0].astype(_i32)
    eap = jnp.concatenate(
        [edge_attr.astype(_f32),
         jnp.ones((_E, 1), _f32),
         jnp.zeros((_E, 6), _f32)], axis=1)

    epack = eap.reshape(_E // 8, 8 * 16)
    acc16 = _sc_edge_stats(dst, src, epack, xi)
    acc = acc16[:_N]

    xf = xi.astype(_f32).reshape(_N, 1)
    sltf = jnp.asarray(self_loop_type, _f32)
    sla16 = jnp.zeros((1, 16), _f32).at[0, self_loop_index].set(sltf)
    zpad = jnp.zeros((16 - _DE, _H), _f32)
    We16_0 = jnp.concatenate([We0.astype(_f32), zpad], axis=0)
    We16_1 = jnp.concatenate([We1.astype(_f32), zpad], axis=0)

    z0, s0, q0 = _tc_a0(acc, xf, embed, sla16, We16_0,
                        be0.reshape(1, _H), W10, b10.reshape(1, 2 * _H))
    h1 = _tc_b(z0, s0, q0, g0.reshape(1, -1), bt0.reshape(1, -1),
               W20, b20.reshape(1, -1), final_relu=True)

    p = _sc_spmm(dst, src, h1)

    z1, s1, q1 = _tc_a1(p[:_N], h1, acc, sla16, We16_1,
                        be1.reshape(1, _H), W11, b11.reshape(1, 2 * _H))
    out = _tc_b(z1, s1, q1, g1.reshape(1, -1), bt1.reshape(1, -1),
                W21, b21.reshape(1, -1), final_relu=False)
    return out
